# baseline (speedup 1.0000x reference)
import functools

import jax
import jax.numpy as jnp
import numpy as np
from jax.experimental import pallas as pl
from jax.experimental.pallas import tpu as pltpu

_FOLD = np.array([[[1, 0, 0], [0, 1, 1]],
                  [[1, 1, 0], [0, 0, 1]]], dtype=np.float32)


def _fold_weights_t(w_oihw):
    fold = jnp.asarray(_FOLD)
    return jnp.einsum("apk,bql,oikl->abpqoi", fold, fold, w_oihw)


def _conv_body(x0_ref, x1_ref, x2_ref, w_ref, b_ref, o_ref, *, H, W, Cin, Cout):
    M = H * W
    bias_v = b_ref[...]

    xq = (x0_ref, x1_ref, x2_ref)
    win = {}
    for q in range(3):
        for p in range(3):
            win[(p, q)] = xq[q][0, :, p * W:p * W + M]

    for a in range(2):
        for b in range(2):
            acc = None
            for dy in range(2):
                for dx in range(2):
                    d = jnp.dot(w_ref[a, b, dy, dx], win[(a + dy, b + dx)],
                                preferred_element_type=jnp.float32)
                    acc = d if acc is None else acc + d
            o_ref[0, a, b] = (acc + bias_v).astype(o_ref.dtype)


def kernel(x_nchw, conv_weight_oihw, conv_bias):
    N, C, H, W = x_nchw.shape
    Cout = conv_weight_oihw.shape[0]
    M = H * W

    xpad = jnp.pad(x_nchw.astype(jnp.bfloat16),
                   ((0, 0), (0, 0), (1, 1), (1, 1)))
    xqs = [xpad[:, :, :, q:q + W].reshape(N, C, (H + 2) * W) for q in range(3)]

    w_t = _fold_weights_t(conv_weight_oihw).astype(jnp.bfloat16)
    bias2 = conv_bias.reshape(Cout, 1).astype(jnp.float32)

    body = functools.partial(_conv_body, H=H, W=W, Cin=C, Cout=Cout)
    xq_spec = pl.BlockSpec((1, C, (H + 2) * W), lambda n: (n, 0, 0))
    y_planes = pl.pallas_call(
        body,
        out_shape=jax.ShapeDtypeStruct((N, 2, 2, Cout, M), jnp.bfloat16),
        grid=(N,),
        in_specs=[
            xq_spec, xq_spec, xq_spec,
            pl.BlockSpec((2, 2, 2, 2, Cout, C), lambda n: (0, 0, 0, 0, 0, 0)),
            pl.BlockSpec((Cout, 1), lambda n: (0, 0)),
        ],
        out_specs=pl.BlockSpec((1, 2, 2, Cout, M), lambda n: (n, 0, 0, 0, 0)),
        compiler_params=pltpu.CompilerParams(
            dimension_semantics=("parallel",)),
        cost_estimate=pl.CostEstimate(
            flops=int(2 * 16 * N * M * C * Cout),
            transcendentals=0,
            bytes_accessed=int(N * C * (3 * (H + 2) * W * 2 + 4 * M * 2)),
        ),
    )(*xqs, w_t, bias2)

    y = y_planes.reshape(N, 2, 2, Cout, H, W)
    y = jnp.transpose(y, (0, 3, 4, 1, 5, 2)).astype(jnp.float32)
    return y.reshape(N, Cout, 2 * H, 2 * W)
